# baseline (speedup 1.0000x reference)
import functools

import jax
import jax.numpy as jnp
from jax import lax
from jax.experimental import pallas as pl
from jax.experimental.pallas import tpu as pltpu
from jax.experimental.pallas import tpu_sc as plsc

NUM_TOKENS = 16
VOCAB = 100000
SPLIT_DIM = 8

NUM_CORES = 2
NUM_SUBCORES = 16
NUM_WORKERS = NUM_CORES * NUM_SUBCORES
LANES = 16

CHUNK = 4096
ADD_UNROLL = 4


def _make_lookup(batch, seq, num_tok, vocab, d):
    n = batch * seq * num_tok
    per_w = n // NUM_WORKERS
    n_chunks = per_w // CHUNK
    assert n % NUM_WORKERS == 0 and per_w % CHUNK == 0
    assert n_chunks % 2 == 1

    mesh = plsc.VectorSubcoreMesh(core_axis_name="c", subcore_axis_name="s")

    @functools.partial(
        pl.kernel,
        mesh=mesh,
        compiler_params=pltpu.CompilerParams(use_tc_tiling_on_sc=False),
        out_type=jax.ShapeDtypeStruct((n, d), jnp.float32),
        scratch_types=[
            pltpu.VMEM((CHUNK,), jnp.int32),
            pltpu.VMEM((CHUNK,), jnp.int32),
            pltpu.VMEM((CHUNK, d), jnp.float32),
            pltpu.VMEM((CHUNK, d), jnp.float32),
            pltpu.SemaphoreType.DMA,
            pltpu.SemaphoreType.DMA,
        ],
    )
    def lookup(x_2d, tab_3d, out_hbm, idx0, idx1, rows0, rows1, s0, s1):
        tab_hbm = tab_3d.at[0]
        wid = lax.axis_index("s") * NUM_CORES + lax.axis_index("c")
        base_w = wid * per_w
        chunk_w = wid * n_chunks
        offs = lax.iota(jnp.int32, LANES) * VOCAB

        def load_and_offset(k, idx_v):
            pltpu.sync_copy(x_2d.at[chunk_w + k], idx_v)

            def add_body(j, c):
                for u in range(ADD_UNROLL):
                    sl = pl.ds((j * ADD_UNROLL + u) * LANES, LANES)
                    idx_v[sl] = idx_v[sl] + offs
                return c

            lax.fori_loop(0, CHUNK // (LANES * ADD_UNROLL), add_body, 0)

        def store_out(k, rows_v):
            pltpu.sync_copy(rows_v, out_hbm.at[pl.ds(base_w + k * CHUNK, CHUNK)])

        load_and_offset(0, idx0)
        pltpu.async_copy(tab_hbm.at[idx0], rows0, s0)

        def pair_body(p, c):
            k0 = 2 * p
            load_and_offset(k0 + 1, idx1)
            pltpu.async_copy(tab_hbm.at[idx1], rows1, s1)
            pltpu.make_async_copy(tab_hbm.at[idx0], rows0, s0).wait()
            store_out(k0, rows0)
            load_and_offset(k0 + 2, idx0)
            pltpu.async_copy(tab_hbm.at[idx0], rows0, s0)
            pltpu.make_async_copy(tab_hbm.at[idx1], rows1, s1).wait()
            store_out(k0 + 1, rows1)
            return c

        lax.fori_loop(0, (n_chunks - 1) // 2, pair_body, 0)

        pltpu.make_async_copy(tab_hbm.at[idx0], rows0, s0).wait()
        store_out(n_chunks - 1, rows0)

    return lookup


def kernel(x, tables):
    batch, seq, num_tok = x.shape
    _, vocab, d = tables.shape
    n = batch * seq * num_tok
    x_2d = x.reshape(n // CHUNK, CHUNK)
    out = _make_lookup(batch, seq, num_tok, vocab, d)(x_2d, tables)
    return out.reshape(batch, seq, num_tok * d)


# speedup vs baseline: 1.0010x; 1.0010x over previous
import functools

import jax
import jax.numpy as jnp
from jax import lax
from jax.experimental import pallas as pl
from jax.experimental.pallas import tpu as pltpu
from jax.experimental.pallas import tpu_sc as plsc

NUM_TOKENS = 16
VOCAB = 100000
SPLIT_DIM = 8

NUM_CORES = 2
NUM_SUBCORES = 16
NUM_WORKERS = NUM_CORES * NUM_SUBCORES
LANES = 16

CHUNK = 4096
ADD_UNROLL = 8


def _make_lookup(batch, seq, num_tok, vocab, d):
    n = batch * seq * num_tok
    per_w = n // NUM_WORKERS
    n_chunks = per_w // CHUNK
    assert n % NUM_WORKERS == 0 and per_w % CHUNK == 0
    assert n_chunks % 2 == 1 and n_chunks >= 3

    mesh = plsc.VectorSubcoreMesh(core_axis_name="c", subcore_axis_name="s")

    @functools.partial(
        pl.kernel,
        mesh=mesh,
        compiler_params=pltpu.CompilerParams(use_tc_tiling_on_sc=False),
        out_type=jax.ShapeDtypeStruct((n, d), jnp.float32),
        scratch_types=[
            pltpu.VMEM((CHUNK,), jnp.int32),
            pltpu.VMEM((CHUNK,), jnp.int32),
            pltpu.VMEM((CHUNK, d), jnp.float32),
            pltpu.VMEM((CHUNK, d), jnp.float32),
            pltpu.SemaphoreType.DMA,
            pltpu.SemaphoreType.DMA,
            pltpu.SemaphoreType.DMA,
            pltpu.SemaphoreType.DMA,
        ],
    )
    def lookup(x_2d, tab_3d, out_hbm, idx0, idx1, rows0, rows1,
               sg0, sg1, st0, st1):
        tab_hbm = tab_3d.at[0]
        wid = lax.axis_index("s") * NUM_CORES + lax.axis_index("c")
        base_w = wid * per_w
        chunk_w = wid * n_chunks
        offs = lax.iota(jnp.int32, LANES) * VOCAB

        def load_and_offset(k, idx_v):
            pltpu.sync_copy(x_2d.at[chunk_w + k], idx_v)

            def add_body(j, c):
                for u in range(ADD_UNROLL):
                    sl = pl.ds((j * ADD_UNROLL + u) * LANES, LANES)
                    idx_v[sl] = idx_v[sl] + offs
                return c

            lax.fori_loop(0, CHUNK // (LANES * ADD_UNROLL), add_body, 0)

        def out_at(k):
            return out_hbm.at[pl.ds(base_w + k * CHUNK, CHUNK)]

        def gather(idx_v, rows_v, sem):
            pltpu.async_copy(tab_hbm.at[idx_v], rows_v, sem)

        def wait_gather(idx_v, rows_v, sem):
            pltpu.make_async_copy(tab_hbm.at[idx_v], rows_v, sem).wait()

        def astore(k, rows_v, sem):
            pltpu.async_copy(rows_v, out_at(k), sem)

        def wait_store(k, rows_v, sem):
            pltpu.make_async_copy(rows_v, out_at(k), sem).wait()

        load_and_offset(0, idx0)
        gather(idx0, rows0, sg0)
        load_and_offset(1, idx1)
        gather(idx1, rows1, sg1)

        def pair_body(p, c):
            k0 = 2 * p
            wait_gather(idx0, rows0, sg0)
            astore(k0, rows0, st0)
            load_and_offset(k0 + 2, idx0)
            wait_store(k0, rows0, st0)
            gather(idx0, rows0, sg0)
            wait_gather(idx1, rows1, sg1)
            astore(k0 + 1, rows1, st1)
            load_and_offset(k0 + 3, idx1)
            wait_store(k0 + 1, rows1, st1)
            gather(idx1, rows1, sg1)
            return c

        lax.fori_loop(0, (n_chunks - 3) // 2, pair_body, 0)

        kl = n_chunks - 3
        wait_gather(idx0, rows0, sg0)
        astore(kl, rows0, st0)
        load_and_offset(kl + 2, idx0)
        wait_store(kl, rows0, st0)
        gather(idx0, rows0, sg0)
        wait_gather(idx1, rows1, sg1)
        astore(kl + 1, rows1, st1)
        wait_gather(idx0, rows0, sg0)
        pltpu.sync_copy(rows0, out_at(kl + 2))
        wait_store(kl + 1, rows1, st1)

    return lookup


def kernel(x, tables):
    batch, seq, num_tok = x.shape
    _, vocab, d = tables.shape
    n = batch * seq * num_tok
    x_2d = x.reshape(n // CHUNK, CHUNK)
    out = _make_lookup(batch, seq, num_tok, vocab, d)(x_2d, tables)
    return out.reshape(batch, seq, num_tok * d)
